# initial kernel scaffold (unmeasured)
import jax
import jax.numpy as jnp
from jax import lax
from jax.experimental import pallas as pl
from jax.experimental.pallas import tpu as pltpu


def kernel(
    x,
):
    def body(*refs):
        pass

    out_shape = jax.ShapeDtypeStruct(..., jnp.float32)
    return pl.pallas_call(body, out_shape=out_shape)(...)


# baseline (device time: 418160 ns/iter reference)
import jax
import jax.numpy as jnp
from jax import lax
from jax.experimental import pallas as pl
from jax.experimental.pallas import tpu as pltpu


def kernel(x):
    m, n = x.shape
    x = x.astype(jnp.bfloat16)

    def body(x_ref, out_ref, rbuf, local_sem, send_sems, recv_sems):
        my_x = lax.axis_index("x")
        my_y = lax.axis_index("y")
        x_nbr = (1 - my_x, my_y)
        y_nbr = (my_x, 1 - my_y)

        barrier_sem = pltpu.get_barrier_semaphore()
        for nbr in (x_nbr, y_nbr):
            pl.semaphore_signal(
                barrier_sem, inc=1,
                device_id=nbr, device_id_type=pl.DeviceIdType.MESH,
            )
        pl.semaphore_wait(barrier_sem, 2)

        col0 = my_y * n

        rdma_x = pltpu.make_async_remote_copy(
            src_ref=x_ref,
            dst_ref=rbuf,
            send_sem=send_sems.at[0],
            recv_sem=recv_sems.at[0],
            device_id=x_nbr,
            device_id_type=pl.DeviceIdType.MESH,
        )
        rdma_x.start()
        local = pltpu.make_async_copy(
            x_ref, out_ref.at[:, pl.ds(col0, n)], local_sem
        )
        local.start()
        local.wait()
        rdma_x.wait()

        out_ref[:, pl.ds(col0, n)] = out_ref[:, pl.ds(col0, n)] + rbuf[:, :]

        rdma_y = pltpu.make_async_remote_copy(
            src_ref=out_ref.at[:, pl.ds(col0, n)],
            dst_ref=out_ref.at[:, pl.ds(col0, n)],
            send_sem=send_sems.at[1],
            recv_sem=recv_sems.at[1],
            device_id=y_nbr,
            device_id_type=pl.DeviceIdType.MESH,
        )
        rdma_y.start()
        rdma_y.wait()

    return pl.pallas_call(
        body,
        out_shape=jax.ShapeDtypeStruct((m, 2 * n), jnp.bfloat16),
        in_specs=[pl.BlockSpec(memory_space=pl.ANY)],
        out_specs=pl.BlockSpec(memory_space=pltpu.VMEM),
        scratch_shapes=[
            pltpu.VMEM((m, n), jnp.bfloat16),
            pltpu.SemaphoreType.DMA,
            pltpu.SemaphoreType.DMA((2,)),
            pltpu.SemaphoreType.DMA((2,)),
        ],
        compiler_params=pltpu.CompilerParams(
            collective_id=0,
            vmem_limit_bytes=100 * 1024 * 1024,
        ),
    )(x)


# device time: 248155 ns/iter; 1.6851x vs baseline; 1.6851x over previous
import jax
import jax.numpy as jnp
from jax import lax
from jax.experimental import pallas as pl
from jax.experimental.pallas import tpu as pltpu

N_CHUNKS = 16


def kernel(x):
    m, n = x.shape
    x = x.astype(jnp.bfloat16)
    assert m % N_CHUNKS == 0
    rows = m // N_CHUNKS

    def body(x_ref, out_ref, rbuf, local_sem,
             send_x, recv_x, send_y, recv_y):
        my_x = lax.axis_index("x")
        my_y = lax.axis_index("y")
        x_nbr = (1 - my_x, my_y)
        y_nbr = (my_x, 1 - my_y)
        col0 = my_y * n

        barrier_sem = pltpu.get_barrier_semaphore()
        for nbr in (x_nbr, y_nbr):
            pl.semaphore_signal(
                barrier_sem, inc=1,
                device_id=nbr, device_id_type=pl.DeviceIdType.MESH,
            )
        pl.semaphore_wait(barrier_sem, 2)

        local = pltpu.make_async_copy(
            x_ref, out_ref.at[:, pl.ds(col0, n)], local_sem
        )
        local.start()

        rdmas_x = []
        for c in range(N_CHUNKS):
            r = pltpu.make_async_remote_copy(
                src_ref=x_ref.at[pl.ds(c * rows, rows), :],
                dst_ref=rbuf.at[pl.ds(c * rows, rows), :],
                send_sem=send_x.at[c],
                recv_sem=recv_x.at[c],
                device_id=x_nbr,
                device_id_type=pl.DeviceIdType.MESH,
            )
            r.start()
            rdmas_x.append(r)

        local.wait()

        rdmas_y = []
        for c in range(N_CHUNKS):
            rdmas_x[c].wait_recv()
            rsl = pl.ds(c * rows, rows)
            out_ref[rsl, pl.ds(col0, n)] = (
                out_ref[rsl, pl.ds(col0, n)] + rbuf[rsl, :]
            )
            r = pltpu.make_async_remote_copy(
                src_ref=out_ref.at[rsl, pl.ds(col0, n)],
                dst_ref=out_ref.at[rsl, pl.ds(col0, n)],
                send_sem=send_y.at[c],
                recv_sem=recv_y.at[c],
                device_id=y_nbr,
                device_id_type=pl.DeviceIdType.MESH,
            )
            r.start()
            rdmas_y.append(r)

        for c in range(N_CHUNKS):
            rdmas_x[c].wait_send()
            rdmas_y[c].wait()

    return pl.pallas_call(
        body,
        out_shape=jax.ShapeDtypeStruct((m, 2 * n), jnp.bfloat16),
        in_specs=[pl.BlockSpec(memory_space=pl.ANY)],
        out_specs=pl.BlockSpec(memory_space=pltpu.VMEM),
        scratch_shapes=[
            pltpu.VMEM((m, n), jnp.bfloat16),
            pltpu.SemaphoreType.DMA,
            pltpu.SemaphoreType.DMA((N_CHUNKS,)),
            pltpu.SemaphoreType.DMA((N_CHUNKS,)),
            pltpu.SemaphoreType.DMA((N_CHUNKS,)),
            pltpu.SemaphoreType.DMA((N_CHUNKS,)),
        ],
        compiler_params=pltpu.CompilerParams(
            collective_id=0,
            vmem_limit_bytes=100 * 1024 * 1024,
        ),
    )(x)


# device time: 232248 ns/iter; 1.8005x vs baseline; 1.0685x over previous
import jax
import jax.numpy as jnp
from jax import lax
from jax.experimental import pallas as pl
from jax.experimental.pallas import tpu as pltpu

N_CHUNKS = 16


def kernel(x):
    m, n = x.shape
    assert m % N_CHUNKS == 0
    rows = m // N_CHUNKS

    def body(x_ref, out_ref, rbuf, staging, local_sems,
             send_x, recv_x, send_y, recv_y):
        my_x = lax.axis_index("x")
        my_y = lax.axis_index("y")
        x_nbr = (1 - my_x, my_y)
        y_nbr = (my_x, 1 - my_y)
        col0 = my_y * n

        barrier_sem = pltpu.get_barrier_semaphore()
        for nbr in (x_nbr, y_nbr):
            pl.semaphore_signal(
                barrier_sem, inc=1,
                device_id=nbr, device_id_type=pl.DeviceIdType.MESH,
            )
        pl.semaphore_wait(barrier_sem, 2)

        def local_dma(c):
            return pltpu.make_async_copy(
                x_ref.at[pl.ds(c * rows, rows), :],
                staging.at[c % 2],
                local_sems.at[c % 2],
            )

        local_dma(0).start()
        local_dma(1).start()

        rdmas_x = []
        for c in range(N_CHUNKS):
            local_dma(c).wait()
            rsl = pl.ds(c * rows, rows)
            out_ref[rsl, pl.ds(col0, n)] = staging[c % 2].astype(jnp.bfloat16)
            if c + 2 < N_CHUNKS:
                local_dma(c + 2).start()
            r = pltpu.make_async_remote_copy(
                src_ref=out_ref.at[rsl, pl.ds(col0, n)],
                dst_ref=rbuf.at[rsl, :],
                send_sem=send_x.at[c],
                recv_sem=recv_x.at[c],
                device_id=x_nbr,
                device_id_type=pl.DeviceIdType.MESH,
            )
            r.start()
            rdmas_x.append(r)

        rdmas_y = []
        for c in range(N_CHUNKS):
            rdmas_x[c].wait_send()
            rdmas_x[c].wait_recv()
            rsl = pl.ds(c * rows, rows)
            out_ref[rsl, pl.ds(col0, n)] = (
                out_ref[rsl, pl.ds(col0, n)] + rbuf[rsl, :]
            )
            r = pltpu.make_async_remote_copy(
                src_ref=out_ref.at[rsl, pl.ds(col0, n)],
                dst_ref=out_ref.at[rsl, pl.ds(col0, n)],
                send_sem=send_y.at[c],
                recv_sem=recv_y.at[c],
                device_id=y_nbr,
                device_id_type=pl.DeviceIdType.MESH,
            )
            r.start()
            rdmas_y.append(r)

        for c in range(N_CHUNKS):
            rdmas_y[c].wait()

    return pl.pallas_call(
        body,
        out_shape=jax.ShapeDtypeStruct((m, 2 * n), jnp.bfloat16),
        in_specs=[pl.BlockSpec(memory_space=pl.ANY)],
        out_specs=pl.BlockSpec(memory_space=pltpu.VMEM),
        scratch_shapes=[
            pltpu.VMEM((m, n), jnp.bfloat16),
            pltpu.VMEM((2, m // N_CHUNKS, n), jnp.float32),
            pltpu.SemaphoreType.DMA((2,)),
            pltpu.SemaphoreType.DMA((N_CHUNKS,)),
            pltpu.SemaphoreType.DMA((N_CHUNKS,)),
            pltpu.SemaphoreType.DMA((N_CHUNKS,)),
            pltpu.SemaphoreType.DMA((N_CHUNKS,)),
        ],
        compiler_params=pltpu.CompilerParams(
            collective_id=0,
            vmem_limit_bytes=100 * 1024 * 1024,
        ),
    )(x)
